# baseline (speedup 1.0000x reference)
import jax
import jax.numpy as jnp
from jax.experimental import pallas as pl
from jax.experimental.pallas import tpu as pltpu
from jax.experimental.pallas import tpu_sc as plsc

_TM = 2048
_TN = 640
_GW = 128
_GC = 128


def _gather_rows(table, idx2d, n_idx):
    mesh = plsc.VectorSubcoreMesh(core_axis_name="c", subcore_axis_name="s")

    @pl.kernel(
        out_type=jax.ShapeDtypeStruct((n_idx, _GC), table.dtype),
        mesh=mesh,
    )
    def gather_kernel(x_hbm, i_hbm, o_hbm):
        def body(i_vmem, o_vmem):
            pltpu.sync_copy(x_hbm.at[i_vmem.at[0]], o_vmem)

        pltpu.emit_pipeline(
            body,
            grid=(n_idx // _GW,),
            in_specs=[pl.BlockSpec((1, _GW), lambda i: (0, i))],
            out_specs=[pl.BlockSpec((_GW, _GC), lambda i: (i, 0))],
            core_axis_name=("c", "s"),
            dimension_semantics=(pltpu.PARALLEL,),
        )(i_hbm, o_hbm)

    return gather_kernel(table, idx2d)


def _mlp_head_kernel(h0_ref, w1_ref, b1_ref, w2_ref, b2_ref, out_ref, h_ref):
    v = pl.program_id(1)

    @pl.when(v == 0)
    def _():
        h0 = h0_ref[...].astype(jnp.bfloat16)
        h = jax.lax.dot_general(
            h0, w1_ref[...], (((1,), (1,)), ((), ())),
            preferred_element_type=jnp.float32,
        )
        h_ref[...] = (h + b1_ref[...]).astype(jnp.bfloat16)

    logits = jax.lax.dot_general(
        h_ref[...], w2_ref[...], (((1,), (1,)), ((), ())),
        preferred_element_type=jnp.float32,
    )
    out_ref[...] = logits + b2_ref[...]


def _mlp_head(h0, w1b, b1_2d, w2b, b2_2d, n_tok, d_model, vocab):
    grid = (n_tok // _TM, vocab // _TN)
    return pl.pallas_call(
        _mlp_head_kernel,
        grid=grid,
        in_specs=[
            pl.BlockSpec((_TM, d_model), lambda t, v: (t, 0)),
            pl.BlockSpec((d_model, d_model), lambda t, v: (0, 0)),
            pl.BlockSpec((1, d_model), lambda t, v: (0, 0)),
            pl.BlockSpec((_TN, d_model), lambda t, v: (v, 0)),
            pl.BlockSpec((1, _TN), lambda t, v: (0, v)),
        ],
        out_specs=pl.BlockSpec((_TM, _TN), lambda t, v: (t, v)),
        out_shape=jax.ShapeDtypeStruct((n_tok, vocab), jnp.float32),
        scratch_shapes=[pltpu.VMEM((_TM, d_model), jnp.bfloat16)],
        compiler_params=pltpu.CompilerParams(
            dimension_semantics=("arbitrary", "arbitrary"),
        ),
    )(h0, w1b, b1_2d, w2b, b2_2d)


def kernel(x, embed, W1, b1, W2, b2):
    b, s = x.shape
    vocab, d_model = embed.shape
    n_tok = b * s
    rows_per_tok = d_model // _GC
    idx = (x.reshape(n_tok, 1).astype(jnp.int32) * rows_per_tok
           + jnp.arange(rows_per_tok, dtype=jnp.int32))
    h0 = _gather_rows(
        embed.reshape(vocab * rows_per_tok, _GC),
        idx.reshape(1, n_tok * rows_per_tok),
        n_tok * rows_per_tok,
    ).reshape(n_tok, d_model)
    logits = _mlp_head(
        h0,
        W1.astype(jnp.bfloat16),
        b1.reshape(1, d_model),
        W2.astype(jnp.bfloat16),
        b2.reshape(1, vocab),
        n_tok, d_model, vocab,
    )
    return logits.reshape(b, s, vocab)


# speedup vs baseline: 1.1652x; 1.1652x over previous
import jax
import jax.numpy as jnp
from jax.experimental import pallas as pl
from jax.experimental.pallas import tpu as pltpu
from jax.experimental.pallas import tpu_sc as plsc

_TM = 2048
_TN = 1280
_TP = 2048
_GW = 128
_GC = 128


def _gather_rows(table, idx2d, n_idx):
    mesh = plsc.VectorSubcoreMesh(core_axis_name="c", subcore_axis_name="s")

    @pl.kernel(
        out_type=jax.ShapeDtypeStruct((n_idx, _GC), table.dtype),
        mesh=mesh,
    )
    def gather_kernel(x_hbm, i_hbm, o_hbm):
        def body(i_vmem, o_vmem):
            pltpu.sync_copy(x_hbm.at[i_vmem.at[0]], o_vmem)

        pltpu.emit_pipeline(
            body,
            grid=(n_idx // _GW,),
            in_specs=[pl.BlockSpec((1, _GW), lambda i: (0, i))],
            out_specs=[pl.BlockSpec((_GW, _GC), lambda i: (i, 0))],
            core_axis_name=("c", "s"),
            dimension_semantics=(pltpu.PARALLEL,),
        )(i_hbm, o_hbm)

    return gather_kernel(table, idx2d)


def _precompute_kernel(w2_ref, w1_ref, b1_ref, b2_ref, m_ref, b2e_ref):
    w2 = w2_ref[...]
    m = jax.lax.dot_general(
        w2, w1_ref[...], (((1,), (0,)), ((), ())),
        preferred_element_type=jnp.float32,
    )
    m_ref[...] = m.astype(jnp.bfloat16)
    b2e_ref[...] = b2_ref[...] + jax.lax.dot_general(
        b1_ref[...], w2, (((1,), (1,)), ((), ())),
        preferred_element_type=jnp.float32,
    )


def _precompute(w2, w1, b1_2d, b2_2d, vocab, d_model):
    return pl.pallas_call(
        _precompute_kernel,
        grid=(vocab // _TP,),
        in_specs=[
            pl.BlockSpec((_TP, d_model), lambda p: (p, 0)),
            pl.BlockSpec((d_model, d_model), lambda p: (0, 0)),
            pl.BlockSpec((1, d_model), lambda p: (0, 0)),
            pl.BlockSpec((1, _TP), lambda p: (0, p)),
        ],
        out_specs=[
            pl.BlockSpec((_TP, d_model), lambda p: (p, 0)),
            pl.BlockSpec((1, _TP), lambda p: (0, p)),
        ],
        out_shape=[
            jax.ShapeDtypeStruct((vocab, d_model), jnp.bfloat16),
            jax.ShapeDtypeStruct((1, vocab), jnp.float32),
        ],
        compiler_params=pltpu.CompilerParams(
            dimension_semantics=("arbitrary",),
        ),
    )(w2, w1, b1_2d, b2_2d)


def _head_kernel(h0_ref, m_ref, b2e_ref, out_ref):
    h0 = h0_ref[...].astype(jnp.bfloat16)
    out_ref[...] = jax.lax.dot_general(
        h0, m_ref[...], (((1,), (1,)), ((), ())),
        preferred_element_type=jnp.float32,
    ) + b2e_ref[...]


def _head(h0, m, b2e, n_tok, d_model, vocab):
    return pl.pallas_call(
        _head_kernel,
        grid=(n_tok // _TM, vocab // _TN),
        in_specs=[
            pl.BlockSpec((_TM, d_model), lambda t, v: (t, 0)),
            pl.BlockSpec((_TN, d_model), lambda t, v: (v, 0)),
            pl.BlockSpec((1, _TN), lambda t, v: (0, v)),
        ],
        out_specs=pl.BlockSpec((_TM, _TN), lambda t, v: (t, v)),
        out_shape=jax.ShapeDtypeStruct((n_tok, vocab), jnp.float32),
        compiler_params=pltpu.CompilerParams(
            dimension_semantics=("arbitrary", "arbitrary"),
        ),
    )(h0, m, b2e)


def kernel(x, embed, W1, b1, W2, b2):
    b, s = x.shape
    vocab, d_model = embed.shape
    n_tok = b * s
    rows_per_tok = d_model // _GC
    idx = (x.reshape(n_tok, 1).astype(jnp.int32) * rows_per_tok
           + jnp.arange(rows_per_tok, dtype=jnp.int32))
    h0 = _gather_rows(
        embed.reshape(vocab * rows_per_tok, _GC),
        idx.reshape(1, n_tok * rows_per_tok),
        n_tok * rows_per_tok,
    ).reshape(n_tok, d_model)
    m, b2e = _precompute(W2, W1, b1.reshape(1, d_model), b2.reshape(1, vocab),
                         vocab, d_model)
    logits = _head(h0, m, b2e, n_tok, d_model, vocab)
    return logits.reshape(b, s, vocab)


# speedup vs baseline: 1.2373x; 1.0618x over previous
import jax
import jax.numpy as jnp
import numpy as np
from jax.experimental import pallas as pl
from jax.experimental.pallas import tpu as pltpu
from jax.experimental.pallas import tpu_sc as plsc

_shard_map = getattr(jax, "shard_map", None)
if _shard_map is None:
    from jax.experimental.shard_map import shard_map as _shard_map

_TM = 2048
_TN = 1280
_TP = 2048
_GW = 128
_GC = 128


def _gather_rows(table, idx2d, n_idx):
    mesh = plsc.VectorSubcoreMesh(core_axis_name="c", subcore_axis_name="s")

    @pl.kernel(
        out_type=jax.ShapeDtypeStruct((n_idx, _GC), table.dtype),
        mesh=mesh,
    )
    def gather_kernel(x_hbm, i_hbm, o_hbm):
        def body(i_vmem, o_vmem):
            pltpu.sync_copy(x_hbm.at[i_vmem.at[0]], o_vmem)

        pltpu.emit_pipeline(
            body,
            grid=(n_idx // _GW,),
            in_specs=[pl.BlockSpec((1, _GW), lambda i: (0, i))],
            out_specs=[pl.BlockSpec((_GW, _GC), lambda i: (i, 0))],
            core_axis_name=("c", "s"),
            dimension_semantics=(pltpu.PARALLEL,),
        )(i_hbm, o_hbm)

    return gather_kernel(table, idx2d)


def _precompute_kernel(w2_ref, w1_ref, b1_ref, b2_ref, m_ref, b2e_ref):
    w2 = w2_ref[...]
    m = jax.lax.dot_general(
        w2, w1_ref[...], (((1,), (0,)), ((), ())),
        preferred_element_type=jnp.float32,
    )
    m_ref[...] = m.astype(jnp.bfloat16)
    b2e_ref[...] = b2_ref[...] + jax.lax.dot_general(
        b1_ref[...], w2, (((1,), (1,)), ((), ())),
        preferred_element_type=jnp.float32,
    )


def _precompute(w2, w1, b1_2d, b2_2d, vocab, d_model):
    return pl.pallas_call(
        _precompute_kernel,
        grid=(vocab // _TP,),
        in_specs=[
            pl.BlockSpec((_TP, d_model), lambda p: (p, 0)),
            pl.BlockSpec((d_model, d_model), lambda p: (0, 0)),
            pl.BlockSpec((1, d_model), lambda p: (0, 0)),
            pl.BlockSpec((1, _TP), lambda p: (0, p)),
        ],
        out_specs=[
            pl.BlockSpec((_TP, d_model), lambda p: (p, 0)),
            pl.BlockSpec((1, _TP), lambda p: (0, p)),
        ],
        out_shape=[
            jax.ShapeDtypeStruct((vocab, d_model), jnp.bfloat16),
            jax.ShapeDtypeStruct((1, vocab), jnp.float32),
        ],
        compiler_params=pltpu.CompilerParams(
            dimension_semantics=("arbitrary",),
        ),
    )(w2, w1, b1_2d, b2_2d)


def _head_kernel(h0_ref, m_ref, b2e_ref, out_ref):
    h0 = h0_ref[...].astype(jnp.bfloat16)
    out_ref[...] = jax.lax.dot_general(
        h0, m_ref[...], (((1,), (1,)), ((), ())),
        preferred_element_type=jnp.float32,
    ) + b2e_ref[...]


def _head(h0, m, b2e, n_tok, d_model, vocab):
    return pl.pallas_call(
        _head_kernel,
        grid=(n_tok // _TM, vocab // _TN),
        in_specs=[
            pl.BlockSpec((_TM, d_model), lambda t, v: (t, 0)),
            pl.BlockSpec((_TN, d_model), lambda t, v: (v, 0)),
            pl.BlockSpec((1, _TN), lambda t, v: (0, v)),
        ],
        out_specs=pl.BlockSpec((_TM, _TN), lambda t, v: (t, v)),
        out_shape=jax.ShapeDtypeStruct((n_tok, vocab), jnp.float32),
        compiler_params=pltpu.CompilerParams(
            dimension_semantics=("arbitrary", "arbitrary"),
        ),
    )(h0, m, b2e)


def _local_forward(xs, embed, W1, b1, W2, b2):
    (n_tok,) = xs.shape
    vocab, d_model = embed.shape
    rows_per_tok = d_model // _GC
    idx = (xs.reshape(n_tok, 1).astype(jnp.int32) * rows_per_tok
           + jnp.arange(rows_per_tok, dtype=jnp.int32))
    h0 = _gather_rows(
        embed.reshape(vocab * rows_per_tok, _GC),
        idx.reshape(1, n_tok * rows_per_tok),
        n_tok * rows_per_tok,
    ).reshape(n_tok, d_model)
    m, b2e = _precompute(W2, W1, b1.reshape(1, d_model), b2.reshape(1, vocab),
                         vocab, d_model)
    return _head(h0, m, b2e, n_tok, d_model, vocab)


def kernel(x, embed, W1, b1, W2, b2):
    b, s = x.shape
    vocab, d_model = embed.shape
    n_tok = b * s
    devs = jax.devices()
    n_dev = 2 if (len(devs) >= 2 and n_tok % (2 * _TM) == 0) else 1
    if n_dev == 1:
        logits = _local_forward(x.reshape(n_tok), embed, W1, b1, W2, b2)
        return logits.reshape(b, s, vocab)
    mesh = jax.sharding.Mesh(np.array(devs[:n_dev]), ("d",))
    p = jax.sharding.PartitionSpec
    logits = _shard_map(
        _local_forward,
        mesh=mesh,
        in_specs=(p("d"), p(None, None), p(None, None), p(None),
                  p(None, None), p(None)),
        out_specs=p("d", None),
        check_vma=False,
    )(x.reshape(n_tok), embed, W1, b1, W2, b2)
    return logits.reshape(b, s, vocab)
